# Initial kernel scaffold; baseline (speedup 1.0000x reference)
import jax
import jax.numpy as jnp
from jax.experimental import pallas as pl


def kernel(H, A_hat_indices, A_hat_values, W1, W2):
    raise NotImplementedError("write your pallas kernel here")


# speedup vs baseline: 5.1013x; 5.1013x over previous
import functools

import jax
import jax.numpy as jnp
from jax import lax
from jax.experimental import pallas as pl
from jax.experimental.pallas import tpu as pltpu
from jax.experimental.pallas import tpu_sc as plsc

C = 10000
D = 128
E = 330000

NC = 2
NS = 16
L = 16
NW = NC * NS

CHUNK = 112
E_PAD = NW * CHUNK * 93
PER_W = E_PAD // NW
N_STEPS = PER_W // CHUNK
CP = 10240
ROWS_PER_TILE = CP // NS


def _spmm_body(g_hbm, rows_hbm, cols_hbm, vals_hbm, out_hbm,
               colbuf, rowbuf, valbuf, gat_v, acc, gsem):
    cid = lax.axis_index("c")
    sid = lax.axis_index("s")
    wid = sid * NC + cid

    def zero_row(r, carry):
        for dd in range(D // L):
            gat_v[r, pl.ds(dd * L, L)] = jnp.zeros((L,), jnp.float32)
        return carry

    lax.fori_loop(0, CHUNK, zero_row, 0)

    zbase = sid * ROWS_PER_TILE
    n_full = ROWS_PER_TILE // CHUNK
    rem = ROWS_PER_TILE - n_full * CHUNK
    for i in range(n_full):
        pltpu.sync_copy(gat_v, acc.at[pl.ds(zbase + i * CHUNK, CHUNK)])
    if rem:
        pltpu.sync_copy(gat_v.at[pl.ds(0, rem)],
                        acc.at[pl.ds(zbase + n_full * CHUNK, rem)])

    plsc.subcore_barrier()

    def step(i, carry):
        base = wid * PER_W + i * CHUNK
        pltpu.sync_copy(cols_hbm.at[pl.ds(base, CHUNK)], colbuf)
        pltpu.async_copy(g_hbm.at[colbuf], gat_v, gsem).wait()
        pltpu.sync_copy(vals_hbm.at[pl.ds(base, CHUNK)], valbuf)

        def scale_grp(g, c2):
            vv = valbuf[pl.ds(g * L, L)]
            for j in range(L):
                v = jnp.full((L,), vv[j], jnp.float32)
                r = g * L + j
                for dd in range(D // L):
                    sl = pl.ds(dd * L, L)
                    gat_v[r, sl] = gat_v[r, sl] * v
            return c2

        lax.fori_loop(0, CHUNK // L, scale_grp, 0)

        pltpu.sync_copy(rows_hbm.at[pl.ds(base, CHUNK)], rowbuf)
        pltpu.sync_copy(gat_v, acc.at[rowbuf], add=True)
        return carry

    lax.fori_loop(0, N_STEPS, step, 0)

    plsc.subcore_barrier()

    for i in range(n_full):
        pltpu.sync_copy(acc.at[pl.ds(zbase + i * CHUNK, CHUNK)],
                        out_hbm.at[cid, pl.ds(zbase + i * CHUNK, CHUNK)])
    if rem:
        pltpu.sync_copy(acc.at[pl.ds(zbase + n_full * CHUNK, rem)],
                        out_hbm.at[cid, pl.ds(zbase + n_full * CHUNK, rem)])


_spmm_partials = pl.kernel(
    _spmm_body,
    out_type=jax.ShapeDtypeStruct((NC, CP, D), jnp.float32),
    mesh=plsc.VectorSubcoreMesh(core_axis_name="c", subcore_axis_name="s",
                                num_cores=NC, num_subcores=NS),
    scratch_types=[
        pltpu.VMEM((CHUNK,), jnp.int32),
        pltpu.VMEM((CHUNK,), jnp.int32),
        pltpu.VMEM((CHUNK,), jnp.float32),
        pltpu.VMEM((CHUNK, D), jnp.float32),
        pltpu.VMEM_SHARED((CP, D), jnp.float32),
        pltpu.SemaphoreType.DMA,
    ],
)


_BLK = 2000


def _mm_tc(x_ref, w_ref, o_ref):
    o_ref[...] = jnp.dot(x_ref[...], w_ref[...],
                         preferred_element_type=jnp.float32)


def _fuse_tc(p_ref, w_ref, o_ref):
    x = jnp.maximum(p_ref[0] + p_ref[1], 0.0)
    o_ref[...] = jnp.dot(x, w_ref[...], preferred_element_type=jnp.float32)


def _addp_tc(q_ref, o_ref):
    o_ref[...] = q_ref[0] + q_ref[1]


def _matmul(x, w):
    return pl.pallas_call(
        _mm_tc,
        grid=(C // _BLK,),
        in_specs=[pl.BlockSpec((_BLK, D), lambda i: (i, 0)),
                  pl.BlockSpec((D, D), lambda i: (0, 0))],
        out_specs=pl.BlockSpec((_BLK, D), lambda i: (i, 0)),
        out_shape=jax.ShapeDtypeStruct((C, D), jnp.float32),
    )(x, w)


def _relu_add_matmul(p, w):
    return pl.pallas_call(
        _fuse_tc,
        grid=(C // _BLK,),
        in_specs=[pl.BlockSpec((NC, _BLK, D), lambda i: (0, i, 0)),
                  pl.BlockSpec((D, D), lambda i: (0, 0))],
        out_specs=pl.BlockSpec((_BLK, D), lambda i: (i, 0)),
        out_shape=jax.ShapeDtypeStruct((C, D), jnp.float32),
    )(p, w)


def _add_partials(q):
    return pl.pallas_call(
        _addp_tc,
        grid=(C // _BLK,),
        in_specs=[pl.BlockSpec((NC, _BLK, D), lambda i: (0, i, 0))],
        out_specs=pl.BlockSpec((_BLK, D), lambda i: (i, 0)),
        out_shape=jax.ShapeDtypeStruct((C, D), jnp.float32),
    )(q)


def kernel(H, A_hat_indices, A_hat_values, W1, W2):
    pad = E_PAD - E
    rows = jnp.pad(A_hat_indices[0], (0, pad))
    cols = jnp.pad(A_hat_indices[1], (0, pad))
    vals = jnp.pad(A_hat_values, (0, pad))

    g1 = _matmul(H, W1)
    p = _spmm_partials(g1, rows, cols, vals)
    g2 = _relu_add_matmul(p, W2)
    q = _spmm_partials(g2, rows, cols, vals)
    return _add_partials(q)


# speedup vs baseline: 5.2250x; 1.0242x over previous
import functools

import jax
import jax.numpy as jnp
from jax import lax
from jax.experimental import pallas as pl
from jax.experimental.pallas import tpu as pltpu
from jax.experimental.pallas import tpu_sc as plsc

C = 10000
D = 128
E = 330000

NC = 2
NS = 16
L = 16
NW = NC * NS

CHUNK = 128
N_STEPS = 81
E_PAD = NW * CHUNK * N_STEPS
PER_W = E_PAD // NW
S_TOTAL = NW * N_STEPS
CP = 10240
ROWS_PER_TILE = CP // NS
N_OUT = ROWS_PER_TILE // CHUNK


def _scale_chunk(gat, b, vbuf, s):
    def scale_grp(g, c2):
        vv = vbuf[s, pl.ds(g * L, L)]
        for j in range(L):
            v = jnp.full((L,), vv[j], jnp.float32)
            r = g * L + j
            for dd in range(D // L):
                sl = pl.ds(dd * L, L)
                gat[b, r, sl] = gat[b, r, sl] * v
        return c2

    lax.fori_loop(0, CHUNK // L, scale_grp, 0)


def _spmm_body(g_hbm, packed_hbm, vals_hbm, out_hbm, ebuf, vbuf, gat, acc,
               esem, gsem, ssem, zsem):
    cid = lax.axis_index("c")
    sid = lax.axis_index("s")
    wid = sid * NC + cid
    cbase = wid * N_STEPS

    pltpu.async_copy(packed_hbm.at[cbase], ebuf.at[0], esem)
    pltpu.async_copy(vals_hbm.at[cbase], vbuf.at[0], esem)

    def zero_row(r, carry):
        for dd in range(D // L):
            gat[0, r, pl.ds(dd * L, L)] = jnp.zeros((L,), jnp.float32)
        return carry

    lax.fori_loop(0, CHUNK, zero_row, 0)

    zbase = sid * ROWS_PER_TILE
    for k in range(N_OUT):
        pltpu.async_copy(gat.at[0], acc.at[pl.ds(zbase + k * CHUNK, CHUNK)],
                         zsem)
    for k in range(N_OUT):
        pltpu.make_async_copy(gat.at[0], acc.at[pl.ds(zbase, CHUNK)],
                              zsem).wait()
    plsc.subcore_barrier()

    def step(i, carry):
        b = i % 2
        nb = 1 - b
        s = i % 3
        ps = (i + 2) % 3

        pltpu.make_async_copy(packed_hbm.at[cbase + i], ebuf.at[s],
                              esem).wait()
        pltpu.make_async_copy(vals_hbm.at[cbase + i], vbuf.at[s],
                              esem).wait()
        @pl.when(i >= 2)
        def _():
            pltpu.make_async_copy(gat.at[b], acc.at[ebuf.at[s, 1]],
                                  ssem).wait()
        @pl.when(i >= 1)
        def _():
            pltpu.make_async_copy(g_hbm.at[ebuf.at[ps, 0]], gat.at[nb],
                                  gsem).wait()
        pltpu.async_copy(g_hbm.at[ebuf.at[s, 0]], gat.at[b], gsem)

        @pl.when(i + 1 < N_STEPS)
        def _():
            pltpu.async_copy(packed_hbm.at[cbase + i + 1],
                             ebuf.at[(i + 1) % 3], esem)
            pltpu.async_copy(vals_hbm.at[cbase + i + 1],
                             vbuf.at[(i + 1) % 3], esem)

        @pl.when(i >= 1)
        def _():
            _scale_chunk(gat, nb, vbuf, ps)
            pltpu.async_copy(gat.at[nb], acc.at[ebuf.at[ps, 1]], ssem,
                             add=True)
        return carry

    lax.fori_loop(0, N_STEPS, step, 0)

    lb = (N_STEPS - 1) % 2
    ls = (N_STEPS - 1) % 3
    pltpu.make_async_copy(g_hbm.at[ebuf.at[ls, 0]], gat.at[lb], gsem).wait()
    _scale_chunk(gat, lb, vbuf, ls)
    pltpu.async_copy(gat.at[lb], acc.at[ebuf.at[ls, 1]], ssem, add=True)
    pltpu.make_async_copy(gat.at[0], acc.at[ebuf.at[0, 1]], ssem).wait()
    pltpu.make_async_copy(gat.at[0], acc.at[ebuf.at[0, 1]], ssem).wait()

    plsc.subcore_barrier()

    for k in range(N_OUT):
        pltpu.async_copy(acc.at[pl.ds(zbase + k * CHUNK, CHUNK)],
                         out_hbm.at[cid, pl.ds(zbase + k * CHUNK, CHUNK)],
                         zsem)
    for k in range(N_OUT):
        pltpu.make_async_copy(acc.at[pl.ds(zbase, CHUNK)],
                              out_hbm.at[cid, pl.ds(zbase, CHUNK)],
                              zsem).wait()


_spmm_partials = pl.kernel(
    _spmm_body,
    out_type=jax.ShapeDtypeStruct((NC, CP, D), jnp.float32),
    mesh=plsc.VectorSubcoreMesh(core_axis_name="c", subcore_axis_name="s",
                                num_cores=NC, num_subcores=NS),
    scratch_types=[
        pltpu.VMEM((3, 2, CHUNK), jnp.int32),
        pltpu.VMEM((3, CHUNK), jnp.float32),
        pltpu.VMEM((2, CHUNK, D), jnp.float32),
        pltpu.VMEM_SHARED((CP, D), jnp.float32),
        pltpu.SemaphoreType.DMA,
        pltpu.SemaphoreType.DMA,
        pltpu.SemaphoreType.DMA,
        pltpu.SemaphoreType.DMA,
    ],
)


_BLK = 2000


def _mm_tc(x_ref, w_ref, o_ref):
    o_ref[...] = jnp.dot(x_ref[...], w_ref[...],
                         preferred_element_type=jnp.float32)


def _fuse_tc(p_ref, w_ref, o_ref):
    x = jnp.maximum(p_ref[0] + p_ref[1], 0.0)
    o_ref[...] = jnp.dot(x, w_ref[...], preferred_element_type=jnp.float32)


def _addp_tc(q_ref, o_ref):
    o_ref[...] = q_ref[0] + q_ref[1]


def _matmul(x, w):
    return pl.pallas_call(
        _mm_tc,
        grid=(C // _BLK,),
        in_specs=[pl.BlockSpec((_BLK, D), lambda i: (i, 0)),
                  pl.BlockSpec((D, D), lambda i: (0, 0))],
        out_specs=pl.BlockSpec((_BLK, D), lambda i: (i, 0)),
        out_shape=jax.ShapeDtypeStruct((C, D), jnp.float32),
    )(x, w)


def _relu_add_matmul(p, w):
    return pl.pallas_call(
        _fuse_tc,
        grid=(C // _BLK,),
        in_specs=[pl.BlockSpec((NC, _BLK, D), lambda i: (0, i, 0)),
                  pl.BlockSpec((D, D), lambda i: (0, 0))],
        out_specs=pl.BlockSpec((_BLK, D), lambda i: (i, 0)),
        out_shape=jax.ShapeDtypeStruct((C, D), jnp.float32),
    )(p, w)


def _add_partials(q):
    return pl.pallas_call(
        _addp_tc,
        grid=(C // _BLK,),
        in_specs=[pl.BlockSpec((NC, _BLK, D), lambda i: (0, i, 0))],
        out_specs=pl.BlockSpec((_BLK, D), lambda i: (i, 0)),
        out_shape=jax.ShapeDtypeStruct((C, D), jnp.float32),
    )(q)


def kernel(H, A_hat_indices, A_hat_values, W1, W2):
    pad = E_PAD - E
    cols = jnp.pad(A_hat_indices[1], (0, pad))
    rows = jnp.pad(A_hat_indices[0], (0, pad))
    vals = jnp.pad(A_hat_values, (0, pad)).reshape(S_TOTAL, CHUNK)
    packed = jnp.stack([cols.reshape(S_TOTAL, CHUNK),
                        rows.reshape(S_TOTAL, CHUNK)], axis=1)

    g1 = _matmul(H, W1)
    p = _spmm_partials(g1, packed, vals)
    g2 = _relu_add_matmul(p, W2)
    q = _spmm_partials(g2, packed, vals)
    return _add_partials(q)


# speedup vs baseline: 5.6198x; 1.0756x over previous
import functools

import jax
import jax.numpy as jnp
from jax import lax
from jax.experimental import pallas as pl
from jax.experimental.pallas import tpu as pltpu
from jax.experimental.pallas import tpu_sc as plsc

C = 10000
D = 128
E = 330000

NC = 2
NS = 16
L = 16
NW = NC * NS

CHUNK = 128
N_STEPS = 82
E_PAD = NW * CHUNK * N_STEPS
PER_W = E_PAD // NW
S_TOTAL = NW * N_STEPS
CP = 10240
ROWS_PER_TILE = CP // NS
N_OUT = ROWS_PER_TILE // CHUNK


def _scale_chunk(gat, b, vbuf, s):
    def scale_grp(g, c2):
        vv = vbuf[s, pl.ds(g * L, L)]
        for j in range(L):
            v = jnp.full((L,), vv[j], jnp.float32)
            r = g * L + j
            for dd in range(D // L):
                sl = pl.ds(dd * L, L)
                gat[b, r, sl] = gat[b, r, sl] * v
        return c2

    lax.fori_loop(0, CHUNK // L, scale_grp, 0)


def _spmm_body(g_hbm, packed_hbm, vals_hbm, out_hbm, ebuf, vbuf, gat, acc,
               esem, gsem, ssem, zsem):
    cid = lax.axis_index("c")
    sid = lax.axis_index("s")
    wid = sid * NC + cid
    cbase = wid * N_STEPS

    pltpu.async_copy(packed_hbm.at[cbase], ebuf.at[0], esem)
    pltpu.async_copy(vals_hbm.at[cbase], vbuf.at[0], esem)

    def zero_row(r, carry):
        for dd in range(D // L):
            gat[0, r, pl.ds(dd * L, L)] = jnp.zeros((L,), jnp.float32)
        return carry

    lax.fori_loop(0, CHUNK, zero_row, 0)

    zbase = sid * ROWS_PER_TILE
    for k in range(N_OUT):
        pltpu.async_copy(gat.at[0], acc.at[pl.ds(zbase + k * CHUNK, CHUNK)],
                         zsem)
    for k in range(N_OUT):
        pltpu.make_async_copy(gat.at[0], acc.at[pl.ds(zbase, CHUNK)],
                              zsem).wait()
    plsc.subcore_barrier()

    def sub_step(p, k):
        i = 2 * p + k
        s = i % 4
        ps = (i + 3) % 4
        ns = (i + 1) % 4
        b = k
        nb = 1 - k

        pltpu.make_async_copy(packed_hbm.at[cbase + i], ebuf.at[s],
                              esem).wait()
        pltpu.make_async_copy(vals_hbm.at[cbase + i], vbuf.at[s],
                              esem).wait()
        @pl.when(i >= 2)
        def _():
            pltpu.make_async_copy(gat.at[b], acc.at[ebuf.at[s, 1]],
                                  ssem).wait()
        @pl.when(i >= 1)
        def _():
            pltpu.make_async_copy(g_hbm.at[ebuf.at[ps, 0]], gat.at[nb],
                                  gsem).wait()
        pltpu.async_copy(g_hbm.at[ebuf.at[s, 0]], gat.at[b], gsem)

        @pl.when(i + 1 < N_STEPS)
        def _():
            pltpu.async_copy(packed_hbm.at[cbase + i + 1], ebuf.at[ns], esem)
            pltpu.async_copy(vals_hbm.at[cbase + i + 1], vbuf.at[ns], esem)

        @pl.when(i >= 1)
        def _():
            _scale_chunk(gat, nb, vbuf, ps)
            pltpu.async_copy(gat.at[nb], acc.at[ebuf.at[ps, 1]], ssem,
                             add=True)

    def step_pair(p, carry):
        sub_step(p, 0)
        sub_step(p, 1)
        return carry

    lax.fori_loop(0, N_STEPS // 2, step_pair, 0)

    lb = (N_STEPS - 1) % 2
    ls = (N_STEPS - 1) % 4
    pltpu.make_async_copy(g_hbm.at[ebuf.at[ls, 0]], gat.at[lb], gsem).wait()
    _scale_chunk(gat, lb, vbuf, ls)
    pltpu.async_copy(gat.at[lb], acc.at[ebuf.at[ls, 1]], ssem, add=True)
    pltpu.make_async_copy(gat.at[0], acc.at[ebuf.at[0, 1]], ssem).wait()
    pltpu.make_async_copy(gat.at[0], acc.at[ebuf.at[0, 1]], ssem).wait()

    plsc.subcore_barrier()

    for k in range(N_OUT):
        pltpu.async_copy(acc.at[pl.ds(zbase + k * CHUNK, CHUNK)],
                         out_hbm.at[cid, pl.ds(zbase + k * CHUNK, CHUNK)],
                         zsem)
    for k in range(N_OUT):
        pltpu.make_async_copy(acc.at[pl.ds(zbase, CHUNK)],
                              out_hbm.at[cid, pl.ds(zbase, CHUNK)],
                              zsem).wait()


_spmm_partials = pl.kernel(
    _spmm_body,
    out_type=jax.ShapeDtypeStruct((NC, CP, D), jnp.float32),
    mesh=plsc.VectorSubcoreMesh(core_axis_name="c", subcore_axis_name="s",
                                num_cores=NC, num_subcores=NS),
    scratch_types=[
        pltpu.VMEM((4, 2, CHUNK), jnp.int32),
        pltpu.VMEM((4, CHUNK), jnp.float32),
        pltpu.VMEM((2, CHUNK, D), jnp.float32),
        pltpu.VMEM_SHARED((CP, D), jnp.float32),
        pltpu.SemaphoreType.DMA,
        pltpu.SemaphoreType.DMA,
        pltpu.SemaphoreType.DMA,
        pltpu.SemaphoreType.DMA,
    ],
)


_BLK = 2000


def _mm_tc(x_ref, w_ref, o_ref):
    o_ref[...] = jnp.dot(x_ref[...], w_ref[...],
                         preferred_element_type=jnp.float32)


def _fuse_tc(p_ref, w_ref, o_ref):
    x = jnp.maximum(p_ref[0] + p_ref[1], 0.0)
    o_ref[...] = jnp.dot(x, w_ref[...], preferred_element_type=jnp.float32)


def _addp_tc(q_ref, o_ref):
    o_ref[...] = q_ref[0] + q_ref[1]


def _matmul(x, w):
    return pl.pallas_call(
        _mm_tc,
        grid=(C // _BLK,),
        in_specs=[pl.BlockSpec((_BLK, D), lambda i: (i, 0)),
                  pl.BlockSpec((D, D), lambda i: (0, 0))],
        out_specs=pl.BlockSpec((_BLK, D), lambda i: (i, 0)),
        out_shape=jax.ShapeDtypeStruct((C, D), jnp.float32),
    )(x, w)


def _relu_add_matmul(p, w):
    return pl.pallas_call(
        _fuse_tc,
        grid=(C // _BLK,),
        in_specs=[pl.BlockSpec((NC, _BLK, D), lambda i: (0, i, 0)),
                  pl.BlockSpec((D, D), lambda i: (0, 0))],
        out_specs=pl.BlockSpec((_BLK, D), lambda i: (i, 0)),
        out_shape=jax.ShapeDtypeStruct((C, D), jnp.float32),
    )(p, w)


def _add_partials(q):
    return pl.pallas_call(
        _addp_tc,
        grid=(C // _BLK,),
        in_specs=[pl.BlockSpec((NC, _BLK, D), lambda i: (0, i, 0))],
        out_specs=pl.BlockSpec((_BLK, D), lambda i: (i, 0)),
        out_shape=jax.ShapeDtypeStruct((C, D), jnp.float32),
    )(q)


def kernel(H, A_hat_indices, A_hat_values, W1, W2):
    pad = E_PAD - E
    cols = jnp.pad(A_hat_indices[1], (0, pad))
    rows = jnp.pad(A_hat_indices[0], (0, pad))
    vals = jnp.pad(A_hat_values, (0, pad)).reshape(S_TOTAL, CHUNK)
    packed = jnp.stack([cols.reshape(S_TOTAL, CHUNK),
                        rows.reshape(S_TOTAL, CHUNK)], axis=1)

    g1 = _matmul(H, W1)
    p = _spmm_partials(g1, packed, vals)
    g2 = _relu_add_matmul(p, W2)
    q = _spmm_partials(g2, packed, vals)
    return _add_partials(q)
